# baseline (speedup 1.0000x reference)
import jax
import jax.numpy as jnp
from jax.experimental import pallas as pl
from jax.experimental.pallas import tpu as pltpu

_N = 10000
_D = 512
_BM = 400
_SC = 2000
_NSUP = _N // _SC


def _fused_body(inp_ref, w_ref, adj_ref, w2_ref, gamma_ref, beta_ref,
                out_ref, sup_ref):
    i = pl.program_id(0)

    @pl.when(i < _NSUP)
    def _prologue():
        chunk = jnp.dot(inp_ref[...], w_ref[...],
                        preferred_element_type=jnp.float32)
        sup_ref[pl.ds(i * _SC, _SC), :] = chunk.astype(jnp.bfloat16)

    @pl.when(i >= _NSUP)
    def _main():
        a = adj_ref[...].astype(jnp.bfloat16)
        h = jnp.dot(a, sup_ref[...], preferred_element_type=jnp.float32)
        h = jnp.maximum(h, 0.0).astype(jnp.bfloat16)
        o = jnp.dot(h, w2_ref[...], preferred_element_type=jnp.float32)
        mean = jnp.mean(o, axis=-1, keepdims=True)
        var = jnp.mean(jnp.square(o - mean), axis=-1, keepdims=True)
        o = (o - mean) * jax.lax.rsqrt(var + 1e-5) * gamma_ref[...] + beta_ref[...]
        out_ref[...] = jnp.maximum(o, 0.0)


def kernel(input, adj, weight, weight2, gamma, beta):
    w_bf16 = weight.astype(jnp.bfloat16)
    w2_bf16 = weight2.astype(jnp.bfloat16)
    gamma2d = gamma.reshape(1, _D)
    beta2d = beta.reshape(1, _D)

    out = pl.pallas_call(
        _fused_body,
        grid=(_NSUP + _N // _BM,),
        in_specs=[
            pl.BlockSpec((_SC, _D), lambda i: (jnp.minimum(i, _NSUP - 1), 0)),
            pl.BlockSpec((_D, _D), lambda i: (0, 0)),
            pl.BlockSpec((_BM, _N), lambda i: (jnp.maximum(i - _NSUP, 0), 0)),
            pl.BlockSpec((_D, _D), lambda i: (0, 0)),
            pl.BlockSpec((1, _D), lambda i: (0, 0)),
            pl.BlockSpec((1, _D), lambda i: (0, 0)),
        ],
        out_specs=pl.BlockSpec((_BM, _D), lambda i: (jnp.maximum(i - _NSUP, 0), 0)),
        out_shape=jax.ShapeDtypeStruct((_N, _D), jnp.float32),
        scratch_shapes=[pltpu.VMEM((_N, _D), jnp.bfloat16)],
        compiler_params=pltpu.CompilerParams(
            dimension_semantics=("arbitrary",),
        ),
    )(input, w_bf16, adj, w2_bf16, gamma2d, beta2d)
    return out


# speedup vs baseline: 1.0248x; 1.0248x over previous
import jax
import jax.numpy as jnp
from jax.experimental import pallas as pl
from jax.experimental.pallas import tpu as pltpu

_N = 10000
_D = 512
_BM = 400
_NM = _N // _BM
_SC = 2000
_NSUP = _N // _SC


def _adj_copy(adj_hbm, buf_ref, sem, j, slot):
    return pltpu.make_async_copy(
        adj_hbm.at[pl.ds(j * _BM, _BM), :],
        buf_ref.at[slot],
        sem.at[slot],
    )


def _fused_body(inp_ref, w_ref, w2_ref, gamma_ref, beta_ref, adj_hbm,
                out_ref, sup_ref, buf_ref, sem):
    i = pl.program_id(0)

    @pl.when(i == 0)
    def _kickoff():
        _adj_copy(adj_hbm, buf_ref, sem, 0, 0).start()
        _adj_copy(adj_hbm, buf_ref, sem, 1, 1).start()

    @pl.when(i < _NSUP)
    def _prologue():
        chunk = jnp.dot(inp_ref[...], w_ref[...],
                        preferred_element_type=jnp.float32)
        sup_ref[pl.ds(i * _SC, _SC), :] = chunk.astype(jnp.bfloat16)

    @pl.when(i >= _NSUP)
    def _main():
        j = i - _NSUP
        slot = jax.lax.rem(j, 2)
        _adj_copy(adj_hbm, buf_ref, sem, j, slot).wait()
        a = buf_ref[slot].astype(jnp.bfloat16)
        h = jnp.dot(a, sup_ref[...], preferred_element_type=jnp.float32)
        h = jnp.maximum(h, 0.0).astype(jnp.bfloat16)
        o = jnp.dot(h, w2_ref[...], preferred_element_type=jnp.float32)
        mean = jnp.mean(o, axis=-1, keepdims=True)
        var = jnp.mean(jnp.square(o - mean), axis=-1, keepdims=True)
        o = (o - mean) * jax.lax.rsqrt(var + 1e-5) * gamma_ref[...] + beta_ref[...]
        out_ref[...] = jnp.maximum(o, 0.0)

        @pl.when(j + 2 < _NM)
        def _next():
            _adj_copy(adj_hbm, buf_ref, sem, j + 2, slot).start()


def kernel(input, adj, weight, weight2, gamma, beta):
    w_bf16 = weight.astype(jnp.bfloat16)
    w2_bf16 = weight2.astype(jnp.bfloat16)
    gamma2d = gamma.reshape(1, _D)
    beta2d = beta.reshape(1, _D)

    out = pl.pallas_call(
        _fused_body,
        grid=(_NSUP + _NM,),
        in_specs=[
            pl.BlockSpec((_SC, _D), lambda i: (jnp.minimum(i, _NSUP - 1), 0)),
            pl.BlockSpec((_D, _D), lambda i: (0, 0)),
            pl.BlockSpec((_D, _D), lambda i: (0, 0)),
            pl.BlockSpec((1, _D), lambda i: (0, 0)),
            pl.BlockSpec((1, _D), lambda i: (0, 0)),
            pl.BlockSpec(memory_space=pl.ANY),
        ],
        out_specs=pl.BlockSpec((_BM, _D), lambda i: (jnp.maximum(i - _NSUP, 0), 0)),
        out_shape=jax.ShapeDtypeStruct((_N, _D), jnp.float32),
        scratch_shapes=[
            pltpu.VMEM((_N, _D), jnp.bfloat16),
            pltpu.VMEM((2, _BM, _N), jnp.float32),
            pltpu.SemaphoreType.DMA((2,)),
        ],
        compiler_params=pltpu.CompilerParams(
            dimension_semantics=("arbitrary",),
        ),
    )(input, w_bf16, w2_bf16, gamma2d, beta2d, adj)
    return out
